# Initial kernel scaffold; baseline (speedup 1.0000x reference)
import jax
import jax.numpy as jnp
from jax.experimental import pallas as pl


def kernel(scores):
    raise NotImplementedError("write your pallas kernel here")


# speedup vs baseline: 369.4571x; 369.4571x over previous
import functools

import jax
import jax.numpy as jnp
from jax import lax
from jax.experimental import pallas as pl
from jax.experimental.pallas import tpu as pltpu
from jax.experimental.pallas import tpu_sc as plsc

_PEN = 1.0
_B, _TT, _TS = 128, 64, 128
_NW = 32
_BPW = _B // _NW
_LANES = 16


def _dp_kernel(sc_ref, d_ref, fs_ref):
    f32 = jnp.float32
    lane = lax.broadcasted_iota(jnp.int32, (_B, _TS), 1)
    lf = lane.astype(f32)
    inv = f32(_PEN / _TS)
    p_fwd = inv * ((lf + 1.0) * (lf + 2.0) * 0.5 - 1.0)
    p_bwd = inv * (lf * (lf + 1.0) * 0.5)
    inf = f32(3.0e38)

    def cummin_fwd(z):
        for k in (1, 2, 4, 8, 16, 32, 64):
            sh = pltpu.roll(z, k, axis=1)
            z = jnp.minimum(z, jnp.where(lane >= k, sh, inf))
        return z

    def cummin_rev(z):
        for k in (1, 2, 4, 8, 16, 32, 64):
            sh = pltpu.roll(z, _TS - k, axis=1)
            z = jnp.minimum(z, jnp.where(lane < _TS - k, sh, inf))
        return z

    fs_ref[0] = jnp.where(lane == 0, f32(0.0), p_fwd + inv)

    for t in range(1, _TT + 1):
        w = fs_ref[t - 1] - sc_ref[:, t - 1, :]
        fs_ref[t] = cummin_fwd(w - p_fwd) + p_fwd

    bs_t = inv * (f32(_TS) - lf)
    cs_next = fs_ref[_TT] + bs_t
    cl = -sc_ref[:, 0, 0:1]

    for i in range(_TT):
        t = _TT - 1 - i
        w = bs_t - sc_ref[:, t, :]
        w = jnp.where(lane == _TS - 1, cl, w)
        bs_t = cummin_rev(w + p_bwd) - p_bwd
        cs_t = fs_ref[t] + bs_t
        d_ref[:, t, :] = (cs_next < pltpu.roll(cs_t, _TS - 1, axis=1)
                          ).astype(jnp.int32)
        cs_next = cs_t
        if i + 1 < _TT:
            cl = cl - sc_ref[:, i + 1, 0:1]


def _compute_decisions(scores):
    return pl.pallas_call(
        _dp_kernel,
        out_shape=jax.ShapeDtypeStruct((_B, _TT, _TS), jnp.int32),
        scratch_shapes=[pltpu.VMEM((_TT + 1, _B, _TS), jnp.float32)],
    )(scores)


def _traceback(d_flat):
    mesh = plsc.VectorSubcoreMesh(core_axis_name="c", subcore_axis_name="s")
    dw = _BPW * _TT * _TS
    bw = _BPW * _TT

    @functools.partial(
        pl.kernel,
        out_type=jax.ShapeDtypeStruct((_B, _TT), jnp.int32),
        mesh=mesh,
        scratch_types=[
            pltpu.VMEM((dw,), jnp.int32),
            pltpu.VMEM((_BPW, _TT), jnp.int32),
        ],
        compiler_params=pltpu.CompilerParams(needs_layout_passes=False),
    )
    def k(d_hbm, out_hbm, d_v, best_v):
        wid = lax.axis_index("s") * 2 + lax.axis_index("c")
        pltpu.sync_copy(d_hbm.at[pl.ds(wid * dw, dw)], d_v)

        lane = lax.iota(jnp.int32, _LANES)
        bl = lane & (_BPW - 1)
        lanes_ok = lane < _BPW
        fill = jnp.full((_LANES,), _TS - 1, jnp.int32)
        for i in range(bw // _LANES):
            p = i * _LANES + lane
            plsc.store_scatter(best_v, [p >> 6, p & (_TT - 1)], fill)

        def body(_, carry):
            t, j = carry
            active = lanes_ok & (t < _TT) & (j < _TS - 1)
            tg = jnp.minimum(t, _TT - 1)
            jg = jnp.minimum(j, _TS - 1)
            dval = plsc.load_gather(d_v, [bl * (_TT * _TS) + tg * _TS + jg],
                                    mask=active)
            write = active & (dval != 0)
            plsc.store_scatter(best_v, [bl, tg], j, mask=write)
            t = jnp.where(write, t + 1, t)
            j = jnp.where(active & (dval == 0), j + 1, j)
            return t, j

        z = jnp.zeros((_LANES,), jnp.int32)
        lax.fori_loop(0, _TT + _TS, body, (z, z))
        pltpu.sync_copy(best_v, out_hbm.at[pl.ds(wid * _BPW, _BPW)])

    return k(d_flat)


def kernel(scores):
    d = _compute_decisions(scores)
    return _traceback(d.reshape(_B * _TT * _TS))
